# baseline (speedup 1.0000x reference)
import functools

import jax
import jax.numpy as jnp
from jax import lax
from jax.experimental import pallas as pl
from jax.experimental.pallas import tpu as pltpu
from jax.experimental.pallas import tpu_sc as plsc

E = 8
N_GROUP = 4
TOPK_GROUP = 2
TOP_K = 2
GROUP_SIZE = E // N_GROUP
ROUTED_SCALING = 2.5
NEG = -1e30
N_SHARED_CHUNKS = 4
L = 16
NW = 32


def _logits_kernel(x_ref, gw_ref, out_ref, xbf_ref):
    xb = x_ref[...].astype(jnp.bfloat16)
    xbf_ref[...] = xb
    out_ref[...] = jax.lax.dot_general(
        gw_ref[...], xb, (((1,), (1,)), ((), ())),
        preferred_element_type=jnp.float32)


def _make_sc_routing(t):
    tpw = t // NW
    mesh = plsc.VectorSubcoreMesh(core_axis_name="c", subcore_axis_name="s")

    @functools.partial(
        pl.kernel, mesh=mesh,
        out_type=jax.ShapeDtypeStruct((NW, N_SHARED_CHUNKS + E, tpw),
                                      jnp.float32),
        scratch_types=[
            pltpu.VMEM((E, tpw), jnp.float32),
            pltpu.VMEM((N_SHARED_CHUNKS + E, tpw), jnp.float32),
            pltpu.VMEM((E, L), jnp.float32),
            pltpu.SemaphoreType.DMA,
        ],
    )
    def routing(lg_hbm, bias_hbm, out_hbm, lg_v, w_v, bias_v, sem):
        wid = lax.axis_index("s") * 2 + lax.axis_index("c")
        c1 = pltpu.async_copy(lg_hbm.at[wid], lg_v, sem)
        c2 = pltpu.async_copy(bias_hbm, bias_v, sem)
        c1.wait()
        c2.wait()
        ones = jnp.ones((L,), jnp.float32)
        for k in range(tpw // L):
            ds = pl.ds(k * L, L)
            sc = []
            for e in range(E):
                z = lg_v[e, ds]
                sc.append(1.0 / (1.0 + jnp.exp(-z)))
            sb = [sc[e] + bias_v[e] for e in range(E)]
            g = [sb[GROUP_SIZE * gi] + sb[GROUP_SIZE * gi + 1]
                 for gi in range(N_GROUP)]
            gmask = []
            for gi in range(N_GROUP):
                r = jnp.zeros((L,), jnp.float32)
                for gj in range(N_GROUP):
                    if gj == gi:
                        continue
                    beats = (g[gj] > g[gi]) if gj > gi else (g[gj] >= g[gi])
                    r = r + jnp.where(beats, 1.0, 0.0)
                gmask.append(r < TOPK_GROUP)
            ms = [jnp.where(gmask[e // GROUP_SIZE], sb[e], NEG)
                  for e in range(E)]
            wts = []
            for ei in range(E):
                r = jnp.zeros((L,), jnp.float32)
                for ej in range(E):
                    if ej == ei:
                        continue
                    beats = ((ms[ej] > ms[ei]) if ej > ei
                             else (ms[ej] >= ms[ei]))
                    r = r + jnp.where(beats, 1.0, 0.0)
                wts.append(jnp.where(r < TOP_K, sc[ei], 0.0))
            denom = wts[0]
            for e in range(1, E):
                denom = denom + wts[e]
            denom = denom + 1e-20
            for c in range(N_SHARED_CHUNKS):
                w_v[c, ds] = ones
            for e in range(E):
                w_v[N_SHARED_CHUNKS + e, ds] = (
                    ROUTED_SCALING * (wts[e] / denom))
        pltpu.sync_copy(w_v, out_hbm.at[wid])

    return routing


def _moe_kernel(x_ref, wt_ref, up_ref, down_ref, sup_ref, sdown_ref, out_ref):
    step = pl.program_id(0)
    t = x_ref.shape[0]
    n_sub = 2
    ts = t // n_sub

    def mlp_acc(wu, wd, first, weighted):
        wub = wu.astype(jnp.bfloat16)
        wdb = wd.astype(jnp.bfloat16)
        for s in range(n_sub):
            sl = pl.ds(s * ts, ts)
            h = jax.lax.dot_general(
                x_ref[sl], wub, (((1,), (1,)), ((), ())),
                preferred_element_type=jnp.float32)
            h = jnp.square(jnp.maximum(h, 0.0)).astype(jnp.bfloat16)
            y = jax.lax.dot_general(
                h, wdb, (((1,), (1,)), ((), ())),
                preferred_element_type=jnp.float32)
            if weighted:
                y = wt_ref[0, sl] * y
            if first:
                out_ref[sl, :] = y
            else:
                out_ref[sl, :] += y

    @pl.when(step == 0)
    def _():
        mlp_acc(sup_ref[...], sdown_ref[...], True, False)

    @pl.when(jnp.logical_and(step > 0, step < N_SHARED_CHUNKS))
    def _():
        mlp_acc(sup_ref[...], sdown_ref[...], False, False)

    @pl.when(step >= N_SHARED_CHUNKS)
    def _():
        mlp_acc(up_ref[0], down_ref[0], False, True)


def kernel(hidden_states, gate_w, e_score_correction_bias, w_up, w_down,
           shared_w_up, shared_w_down):
    t, d = hidden_states.shape
    e, d_ff, _ = w_up.shape
    d_ff_sh = shared_w_up.shape[0]
    assert e == E and d_ff_sh == N_SHARED_CHUNKS * d_ff
    n_steps = N_SHARED_CHUNKS + E
    tpw = t // NW

    logits_t, x_bf = pl.pallas_call(
        _logits_kernel,
        out_shape=(jax.ShapeDtypeStruct((E, t), jnp.float32),
                   jax.ShapeDtypeStruct((t, d), jnp.bfloat16)),
    )(hidden_states, gate_w)

    lgb = logits_t.reshape(E, NW, tpw).transpose(1, 0, 2)
    bias16 = jnp.tile(e_score_correction_bias[:, None], (1, L))
    w_sc = _make_sc_routing(t)(lgb, bias16)
    wt3 = w_sc.transpose(1, 0, 2).reshape(n_steps, t, 1)

    nsc = N_SHARED_CHUNKS
    out = pl.pallas_call(
        _moe_kernel,
        grid=(n_steps,),
        in_specs=[
            pl.BlockSpec((t, d), lambda i: (0, 0)),
            pl.BlockSpec((1, t, 1), lambda i: (i, 0, 0)),
            pl.BlockSpec((1, d_ff, d),
                         lambda i: (jnp.maximum(i - nsc, 0), 0, 0)),
            pl.BlockSpec((1, d, d_ff),
                         lambda i: (jnp.maximum(i - nsc, 0), 0, 0)),
            pl.BlockSpec((d_ff, d), lambda i: (jnp.minimum(i, nsc - 1), 0)),
            pl.BlockSpec((d, d_ff), lambda i: (0, jnp.minimum(i, nsc - 1))),
        ],
        out_specs=pl.BlockSpec((t, d), lambda i: (0, 0)),
        out_shape=jax.ShapeDtypeStruct((t, d), jnp.float32),
        compiler_params=pltpu.CompilerParams(
            dimension_semantics=("arbitrary",),
            vmem_limit_bytes=100 * 1024 * 1024,
        ),
    )(x_bf, wt3, w_up, w_down, shared_w_up, shared_w_down)
    return out


# speedup vs baseline: 1.0879x; 1.0879x over previous
import functools

import jax
import jax.numpy as jnp
from jax import lax
from jax.experimental import pallas as pl
from jax.experimental.pallas import tpu as pltpu
from jax.experimental.pallas import tpu_sc as plsc

E = 8
N_GROUP = 4
TOPK_GROUP = 2
TOP_K = 2
GROUP_SIZE = E // N_GROUP
ROUTED_SCALING = 2.5
NEG = -1e30
N_SHARED_CHUNKS = 4
L = 16
NW = 32


def _logits_kernel(x_ref, gw_ref, out_ref, xbf_ref):
    xb = x_ref[...].astype(jnp.bfloat16)
    xbf_ref[...] = xb
    out_ref[...] = jax.lax.dot_general(
        gw_ref[...], xb, (((1,), (1,)), ((), ())),
        preferred_element_type=jnp.float32)


def _make_sc_routing(t):
    tpw = t // NW
    mesh = plsc.VectorSubcoreMesh(core_axis_name="c", subcore_axis_name="s")

    @functools.partial(
        pl.kernel, mesh=mesh,
        out_type=jax.ShapeDtypeStruct((N_SHARED_CHUNKS + E, t), jnp.float32),
        scratch_types=[
            pltpu.VMEM((E, tpw), jnp.float32),
            pltpu.VMEM((N_SHARED_CHUNKS + E, tpw), jnp.float32),
            pltpu.VMEM((E, L), jnp.float32),
            pltpu.SemaphoreType.DMA,
        ],
    )
    def routing(lg_hbm, bias_hbm, out_hbm, lg_v, w_v, bias_v, sem):
        wid = lax.axis_index("s") * 2 + lax.axis_index("c")
        base = wid * tpw
        copies = [pltpu.async_copy(lg_hbm.at[e, pl.ds(base, tpw)],
                                   lg_v.at[e], sem) for e in range(E)]
        copies.append(pltpu.async_copy(bias_hbm, bias_v, sem))
        for c in copies:
            c.wait()
        ones = jnp.ones((L,), jnp.float32)
        for k in range(tpw // L):
            ds = pl.ds(k * L, L)
            sc = []
            for e in range(E):
                z = lg_v[e, ds]
                sc.append(1.0 / (1.0 + jnp.exp(-z)))
            sb = [sc[e] + bias_v[e] for e in range(E)]
            g = [sb[GROUP_SIZE * gi] + sb[GROUP_SIZE * gi + 1]
                 for gi in range(N_GROUP)]
            gmask = []
            for gi in range(N_GROUP):
                r = jnp.zeros((L,), jnp.float32)
                for gj in range(N_GROUP):
                    if gj == gi:
                        continue
                    beats = (g[gj] > g[gi]) if gj > gi else (g[gj] >= g[gi])
                    r = r + jnp.where(beats, 1.0, 0.0)
                gmask.append(r < TOPK_GROUP)
            ms = [jnp.where(gmask[e // GROUP_SIZE], sb[e], NEG)
                  for e in range(E)]
            wts = []
            for ei in range(E):
                r = jnp.zeros((L,), jnp.float32)
                for ej in range(E):
                    if ej == ei:
                        continue
                    beats = ((ms[ej] > ms[ei]) if ej > ei
                             else (ms[ej] >= ms[ei]))
                    r = r + jnp.where(beats, 1.0, 0.0)
                wts.append(jnp.where(r < TOP_K, sc[ei], 0.0))
            denom = wts[0]
            for e in range(1, E):
                denom = denom + wts[e]
            denom = denom + 1e-20
            for c in range(N_SHARED_CHUNKS):
                w_v[c, ds] = ones
            for e in range(E):
                w_v[N_SHARED_CHUNKS + e, ds] = (
                    ROUTED_SCALING * (wts[e] / denom))
        ocopies = [pltpu.async_copy(w_v.at[r], out_hbm.at[r, pl.ds(base, tpw)],
                                    sem) for r in range(N_SHARED_CHUNKS + E)]
        for c in ocopies:
            c.wait()

    return routing


def _moe_kernel(x_ref, wt_ref, up_ref, down_ref, sup_ref, sdown_ref, out_ref):
    step = pl.program_id(0)
    t = x_ref.shape[0]
    n_sub = 2
    ts = t // n_sub

    def mlp_acc(wu, wd, first, weighted):
        wub = wu.astype(jnp.bfloat16)
        wdb = wd.astype(jnp.bfloat16)
        for s in range(n_sub):
            sl = pl.ds(s * ts, ts)
            h = jax.lax.dot_general(
                x_ref[sl], wub, (((1,), (1,)), ((), ())),
                preferred_element_type=jnp.float32)
            h = jnp.square(jnp.maximum(h, 0.0)).astype(jnp.bfloat16)
            y = jax.lax.dot_general(
                h, wdb, (((1,), (1,)), ((), ())),
                preferred_element_type=jnp.float32)
            if weighted:
                y = wt_ref[0, sl] * y
            if first:
                out_ref[sl, :] = y
            else:
                out_ref[sl, :] += y

    @pl.when(step == 0)
    def _():
        mlp_acc(sup_ref[...], sdown_ref[...], True, False)

    @pl.when(jnp.logical_and(step > 0, step < N_SHARED_CHUNKS))
    def _():
        mlp_acc(sup_ref[...], sdown_ref[...], False, False)

    @pl.when(step >= N_SHARED_CHUNKS)
    def _():
        mlp_acc(up_ref[0], down_ref[0], False, True)


def kernel(hidden_states, gate_w, e_score_correction_bias, w_up, w_down,
           shared_w_up, shared_w_down):
    t, d = hidden_states.shape
    e, d_ff, _ = w_up.shape
    d_ff_sh = shared_w_up.shape[0]
    assert e == E and d_ff_sh == N_SHARED_CHUNKS * d_ff
    n_steps = N_SHARED_CHUNKS + E
    tpw = t // NW

    logits_t, x_bf = pl.pallas_call(
        _logits_kernel,
        out_shape=(jax.ShapeDtypeStruct((E, t), jnp.float32),
                   jax.ShapeDtypeStruct((t, d), jnp.bfloat16)),
    )(hidden_states, gate_w)

    bias16 = jnp.tile(e_score_correction_bias[:, None], (1, L))
    w_sc = _make_sc_routing(t)(logits_t, bias16)
    wt3 = w_sc.reshape(n_steps, t, 1)

    nsc = N_SHARED_CHUNKS
    out = pl.pallas_call(
        _moe_kernel,
        grid=(n_steps,),
        in_specs=[
            pl.BlockSpec((t, d), lambda i: (0, 0)),
            pl.BlockSpec((1, t, 1), lambda i: (i, 0, 0)),
            pl.BlockSpec((1, d_ff, d),
                         lambda i: (jnp.maximum(i - nsc, 0), 0, 0)),
            pl.BlockSpec((1, d, d_ff),
                         lambda i: (jnp.maximum(i - nsc, 0), 0, 0)),
            pl.BlockSpec((d_ff, d), lambda i: (jnp.minimum(i, nsc - 1), 0)),
            pl.BlockSpec((d, d_ff), lambda i: (0, jnp.minimum(i, nsc - 1))),
        ],
        out_specs=pl.BlockSpec((t, d), lambda i: (0, 0)),
        out_shape=jax.ShapeDtypeStruct((t, d), jnp.float32),
        compiler_params=pltpu.CompilerParams(
            dimension_semantics=("arbitrary",),
            vmem_limit_bytes=100 * 1024 * 1024,
        ),
    )(x_bf, wt3, w_up, w_down, shared_w_up, shared_w_down)
    return out
